# baseline (speedup 1.0000x reference)
import functools
import math

import jax
import jax.numpy as jnp
from jax import lax
from jax.experimental import pallas as pl
from jax.experimental.pallas import tpu as pltpu
from jax.experimental.pallas import tpu_sc as plsc

_HIDDEN = 64
_SCALE = math.sqrt(float(_HIDDEN))
_LANES = 128
_NC, _NS = 2, 16
_NW = _NC * _NS
_B = 4096 * 200
_ROWS = _B // _LANES
_RPW = _ROWS // _NW
_CR = 4
_CHUNK = _CR * _LANES
_G = _RPW // _CR
_NB = 3


def _emb_body(x_hbm, table_hbm, out_hbm, idx_v, rows_v, gs0, gs1, gs2,
              ss0, ss1, ss2):
    gsems = (gs0, gs1, gs2)
    ssems = (ss0, ss1, ss2)
    wid = lax.axis_index("s") * _NC + lax.axis_index("c")
    rbase = wid * _RPW

    def fire_gather(g, b):
        r = rbase + g * _CR
        pltpu.sync_copy(x_hbm.at[pl.ds(r, _CR)], idx_v.at[b])
        for j in range(_CR):
            pltpu.async_copy(table_hbm.at[idx_v.at[b, j]], rows_v.at[b, j],
                             gsems[b])

    def drain_gather(b):
        for j in range(_CR):
            pltpu.make_async_copy(table_hbm.at[idx_v.at[b, j]],
                                  rows_v.at[b, j], gsems[b]).wait()

    def scale(b):
        for j in range(_CR):
            @plsc.parallel_loop(0, _LANES, unroll=4)
            def _(r):
                for c in range(_HIDDEN // 16):
                    sl = (b, j, r, pl.ds(c * 16, 16))
                    rows_v[sl] = rows_v[sl] * _SCALE

    def fire_scatter(g, b):
        r = rbase + g * _CR
        pltpu.async_copy(rows_v.at[b], out_hbm.at[pl.ds(r, _CR)], ssems[b])

    def drain_scatter(g, b):
        r = rbase + g * _CR
        pltpu.make_async_copy(rows_v.at[b], out_hbm.at[pl.ds(r, _CR)],
                              ssems[b]).wait()

    fire_gather(0, 0)

    @pl.loop(0, _G + (-_G % _NB), step=_NB)
    def _(g0):
        for b in range(_NB):
            g = g0 + b
            nb = (b + 1) % _NB

            @pl.when(g < _G)
            def _():
                @pl.when(g + 1 < _G)
                def _():
                    @pl.when(g >= 2)
                    def _():
                        drain_scatter(g - 2, nb)
                    fire_gather(g + 1, nb)

                drain_gather(b)
                scale(b)
                fire_scatter(g, b)

    for g in range(_G - _NB, _G):
        drain_scatter(g, g % _NB)


@jax.jit
def kernel(x, table):
    xr = x.reshape(_ROWS, _LANES)
    mesh = plsc.VectorSubcoreMesh(core_axis_name="c", subcore_axis_name="s")
    out = pl.kernel(
        _emb_body,
        out_type=jax.ShapeDtypeStruct((_ROWS, _LANES, _HIDDEN), jnp.float32),
        mesh=mesh,
        compiler_params=pltpu.CompilerParams(use_tc_tiling_on_sc=False),
        scratch_types=[
            pltpu.VMEM((_NB, _CR, _LANES), jnp.int32),
            pltpu.VMEM((_NB, _CR, _LANES, _HIDDEN), jnp.float32),
            pltpu.SemaphoreType.DMA,
            pltpu.SemaphoreType.DMA,
            pltpu.SemaphoreType.DMA,
            pltpu.SemaphoreType.DMA,
            pltpu.SemaphoreType.DMA,
            pltpu.SemaphoreType.DMA,
        ],
    )(xr, table)
    return out.reshape(x.shape[0], x.shape[1], _HIDDEN)
